# baseline (speedup 1.0000x reference)
import jax
import jax.numpy as jnp
from jax.experimental import pallas as pl
from jax.experimental.pallas import tpu as pltpu

N_EMB = 1024
E_DIM = 64
COMMIT_COST = 0.25
N_TOK = 32768
ROWS = 1024
GRID = N_TOK // ROWS


def _vq_body(z_ref, emb_ref, enc_ref, zq_ref, idx_ref, psum_ref, pcnt_ref):
    zt = z_ref[0]
    emb = emb_ref[...]

    z2t = jnp.sum(zt * zt, axis=0, keepdims=True)
    z2 = jnp.transpose(z2t, (1, 0))
    e2 = jnp.sum(emb * emb, axis=1, keepdims=True)
    mm2 = jax.lax.dot_general(zt, emb + emb, (((0,), (1,)), ((), ())),
                              preferred_element_type=jnp.float32)
    d = (z2 + e2[:, 0][None, :]) - mm2

    dmin = jnp.min(d, axis=1, keepdims=True)
    colsf = jax.lax.broadcasted_iota(jnp.int32, (ROWS, N_EMB), 1).astype(jnp.float32)
    idxf = jnp.min(jnp.where(d == dmin, colsf, float(N_EMB)), axis=1,
                   keepdims=True)

    oh = jnp.where(colsf == idxf, 1.0, 0.0).astype(jnp.float32)
    enc_ref[...] = oh
    zq = jax.lax.dot_general(oh, emb, (((1,), (0,)), ((), ())),
                             preferred_element_type=jnp.float32)
    zq_ref[...] = zq
    idx_ref[...] = idxf.astype(jnp.int32)

    cross = jax.lax.dot_general(zt, zq, (((1,), (0,)), ((), ())),
                                preferred_element_type=jnp.float32)
    rr = jax.lax.broadcasted_iota(jnp.int32, (E_DIM, E_DIM), 0)
    cc = jax.lax.broadcasted_iota(jnp.int32, (E_DIM, E_DIM), 1)
    tr = jnp.sum(jnp.where(rr == cc, cross, 0.0), axis=(0, 1), keepdims=True)
    sq = (jnp.sum(z2t, axis=(0, 1), keepdims=True)
          + jnp.sum(zq * zq, axis=(0, 1), keepdims=True) - 2.0 * tr)
    psum_ref[...] = sq[:, :, None]
    ones_row = jnp.full((1, ROWS), 1.0, jnp.float32)
    pcnt_ref[...] = jax.lax.dot_general(ones_row, oh, (((1,), (0,)), ((), ())),
                                        preferred_element_type=jnp.float32)[None]


def _finish_body(psum_ref, pcnt_ref, loss_ref, ppl_ref):
    total = jnp.sum(psum_ref[...], axis=(0, 1, 2))
    mse = total / (N_TOK * E_DIM)
    loss_ref[...] = jnp.full((1, 1), 0.0, jnp.float32) + mse * (1.0 + COMMIT_COST)
    e_mean = jnp.sum(pcnt_ref[...], axis=0) / N_TOK
    ent = -jnp.sum(e_mean * jnp.log(e_mean + 1e-10), axis=(0, 1), keepdims=True)
    ppl_ref[...] = jnp.exp(ent)


def _vq_call(z_r, emb):
    enc, zq, idx, psum, pcnt = pl.pallas_call(
        _vq_body,
        grid=(GRID,),
        in_specs=[
            pl.BlockSpec((1, E_DIM, ROWS), lambda i: (i, 0, 0)),
            pl.BlockSpec((N_EMB, E_DIM), lambda i: (0, 0)),
        ],
        out_specs=[
            pl.BlockSpec((ROWS, N_EMB), lambda i: (i, 0)),
            pl.BlockSpec((ROWS, E_DIM), lambda i: (i, 0)),
            pl.BlockSpec((ROWS, 1), lambda i: (i, 0)),
            pl.BlockSpec((1, 1, 1), lambda i: (i, 0, 0)),
            pl.BlockSpec((1, 1, N_EMB), lambda i: (i, 0, 0)),
        ],
        out_shape=[
            jax.ShapeDtypeStruct((N_TOK, N_EMB), jnp.float32),
            jax.ShapeDtypeStruct((N_TOK, E_DIM), jnp.float32),
            jax.ShapeDtypeStruct((N_TOK, 1), jnp.int32),
            jax.ShapeDtypeStruct((GRID, 1, 1), jnp.float32),
            jax.ShapeDtypeStruct((GRID, 1, N_EMB), jnp.float32),
        ],
        compiler_params=pltpu.CompilerParams(
            dimension_semantics=("parallel",),
        ),
    )(z_r, emb)
    loss, ppl = pl.pallas_call(
        _finish_body,
        out_shape=[
            jax.ShapeDtypeStruct((1, 1), jnp.float32),
            jax.ShapeDtypeStruct((1, 1), jnp.float32),
        ],
    )(psum, pcnt)
    return enc, zq, idx, loss, ppl


def kernel(z, emb):
    B = z.shape[0]
    z_r = z.reshape(B, E_DIM, ROWS)
    enc, zq_flat, idx, loss, ppl = _vq_call(z_r, emb)
    H = W = int(ROWS ** 0.5)
    z_q = jnp.transpose(zq_flat.reshape(B, H, W, E_DIM), (0, 2, 3, 1))
    return (loss[0, 0], z_q, ppl[0, 0], enc, idx)


# speedup vs baseline: 1.1307x; 1.1307x over previous
import jax
import jax.numpy as jnp
from jax.experimental import pallas as pl
from jax.experimental.pallas import tpu as pltpu

N_EMB = 1024
E_DIM = 64
COMMIT_COST = 0.25
N_TOK = 32768
ROWS = 512
GRID = N_TOK // ROWS


def _vq_body(z_ref, emb_ref, enc_ref, zq_ref, idx_ref, psum_ref, pcnt_ref):
    z = z_ref[...]
    emb = emb_ref[...]

    z2 = jnp.sum(z * z, axis=1, keepdims=True)
    e2 = jnp.sum(emb * emb, axis=1, keepdims=True)
    mm2 = jax.lax.dot_general(z, emb + emb, (((1,), (1,)), ((), ())),
                              preferred_element_type=jnp.float32)
    d = (z2 + e2[:, 0][None, :]) - mm2

    dmin = jnp.min(d, axis=1, keepdims=True)
    colsf = jax.lax.broadcasted_iota(jnp.int32, (ROWS, N_EMB), 1).astype(jnp.float32)
    idxf = jnp.min(jnp.where(d == dmin, colsf, float(N_EMB)), axis=1,
                   keepdims=True)

    oh = jnp.where(colsf == idxf, 1.0, 0.0).astype(jnp.float32)
    enc_ref[...] = oh
    zq = jax.lax.dot_general(oh, emb, (((1,), (0,)), ((), ())),
                             preferred_element_type=jnp.float32)
    zq_ref[...] = zq
    idx_ref[...] = idxf.astype(jnp.int32)

    diff = zq - z
    psum_ref[...] = jnp.sum(diff * diff, axis=(0, 1), keepdims=True)[:, :, None]
    ones_row = jnp.full((1, ROWS), 1.0, jnp.float32)
    pcnt_ref[...] = jax.lax.dot_general(ones_row, oh, (((1,), (0,)), ((), ())),
                                        preferred_element_type=jnp.float32)[None]


def _finish_body(psum_ref, pcnt_ref, loss_ref, ppl_ref):
    total = jnp.sum(psum_ref[...], axis=(0, 1, 2))
    mse = total / (N_TOK * E_DIM)
    loss_ref[...] = jnp.full((1, 1), 0.0, jnp.float32) + mse * (1.0 + COMMIT_COST)
    e_mean = jnp.sum(pcnt_ref[...], axis=0) / N_TOK
    ent = -jnp.sum(e_mean * jnp.log(e_mean + 1e-10), axis=(0, 1), keepdims=True)
    ppl_ref[...] = jnp.exp(ent)


def _vq_call(z_r, emb):
    enc, zq, idx, psum, pcnt = pl.pallas_call(
        _vq_body,
        grid=(GRID,),
        in_specs=[
            pl.BlockSpec((ROWS, E_DIM), lambda i: (i, 0)),
            pl.BlockSpec((N_EMB, E_DIM), lambda i: (0, 0)),
        ],
        out_specs=[
            pl.BlockSpec((ROWS, N_EMB), lambda i: (i, 0)),
            pl.BlockSpec((ROWS, E_DIM), lambda i: (i, 0)),
            pl.BlockSpec((ROWS, 1), lambda i: (i, 0)),
            pl.BlockSpec((1, 1, 1), lambda i: (i, 0, 0)),
            pl.BlockSpec((1, 1, N_EMB), lambda i: (i, 0, 0)),
        ],
        out_shape=[
            jax.ShapeDtypeStruct((N_TOK, N_EMB), jnp.float32),
            jax.ShapeDtypeStruct((N_TOK, E_DIM), jnp.float32),
            jax.ShapeDtypeStruct((N_TOK, 1), jnp.int32),
            jax.ShapeDtypeStruct((GRID, 1, 1), jnp.float32),
            jax.ShapeDtypeStruct((GRID, 1, N_EMB), jnp.float32),
        ],
        compiler_params=pltpu.CompilerParams(
            dimension_semantics=("parallel",),
        ),
    )(z_r, emb)
    loss, ppl = pl.pallas_call(
        _finish_body,
        out_shape=[
            jax.ShapeDtypeStruct((1, 1), jnp.float32),
            jax.ShapeDtypeStruct((1, 1), jnp.float32),
        ],
    )(psum, pcnt)
    return enc, zq, idx, loss, ppl


def kernel(z, emb):
    z_p = jnp.transpose(z, (0, 2, 3, 1))
    z_flat = z_p.reshape(-1, E_DIM)
    enc, zq_flat, idx, loss, ppl = _vq_call(z_flat, emb)
    z_q = jnp.transpose(zq_flat.reshape(z_p.shape), (0, 2, 3, 1))
    return (loss[0, 0], z_q, ppl[0, 0], enc, idx)


# speedup vs baseline: 1.3261x; 1.1729x over previous
import jax
import jax.numpy as jnp
from jax.experimental import pallas as pl
from jax.experimental.pallas import tpu as pltpu

N_EMB = 1024
E_DIM = 64
COMMIT_COST = 0.25
N_TOK = 32768
ROWS = 2048
GRID = N_TOK // ROWS


def _vq_body(z_ref, emb_ref, enc_ref, zq_ref, idx_ref, psum_ref, pcnt_ref):
    z = z_ref[...]
    emb = emb_ref[...]

    z2 = jnp.sum(z * z, axis=1, keepdims=True)
    e2 = jnp.sum(emb * emb, axis=1, keepdims=True)
    mm2 = jax.lax.dot_general(z, emb + emb, (((1,), (1,)), ((), ())),
                              preferred_element_type=jnp.float32)
    d = (z2 + e2[:, 0][None, :]) - mm2

    dmin = jnp.min(d, axis=1, keepdims=True)
    colsf = jax.lax.broadcasted_iota(jnp.int32, (ROWS, N_EMB), 1).astype(jnp.float32)
    idxf = jnp.min(jnp.where(d == dmin, colsf, float(N_EMB)), axis=1,
                   keepdims=True)

    oh = jnp.where(colsf == idxf, 1.0, 0.0).astype(jnp.float32)
    enc_ref[...] = oh
    zq = jax.lax.dot_general(oh, emb, (((1,), (0,)), ((), ())),
                             preferred_element_type=jnp.float32)
    zq_ref[...] = zq
    idx_ref[...] = idxf.astype(jnp.int32)

    diff = zq - z
    psum_ref[...] = jnp.sum(diff * diff, axis=(0, 1), keepdims=True)[:, :, None]
    ones_row = jnp.full((1, ROWS), 1.0, jnp.float32)
    pcnt_ref[...] = jax.lax.dot_general(ones_row, oh, (((1,), (0,)), ((), ())),
                                        preferred_element_type=jnp.float32)[None]


def _finish_body(psum_ref, pcnt_ref, loss_ref, ppl_ref):
    total = jnp.sum(psum_ref[...], axis=(0, 1, 2))
    mse = total / (N_TOK * E_DIM)
    loss_ref[...] = jnp.full((1, 1), 0.0, jnp.float32) + mse * (1.0 + COMMIT_COST)
    e_mean = jnp.sum(pcnt_ref[...], axis=0) / N_TOK
    ent = -jnp.sum(e_mean * jnp.log(e_mean + 1e-10), axis=(0, 1), keepdims=True)
    ppl_ref[...] = jnp.exp(ent)


def _vq_call(z_r, emb):
    enc, zq, idx, psum, pcnt = pl.pallas_call(
        _vq_body,
        grid=(GRID,),
        in_specs=[
            pl.BlockSpec((ROWS, E_DIM), lambda i: (i, 0)),
            pl.BlockSpec((N_EMB, E_DIM), lambda i: (0, 0)),
        ],
        out_specs=[
            pl.BlockSpec((ROWS, N_EMB), lambda i: (i, 0)),
            pl.BlockSpec((ROWS, E_DIM), lambda i: (i, 0)),
            pl.BlockSpec((ROWS, 1), lambda i: (i, 0)),
            pl.BlockSpec((1, 1, 1), lambda i: (i, 0, 0)),
            pl.BlockSpec((1, 1, N_EMB), lambda i: (i, 0, 0)),
        ],
        out_shape=[
            jax.ShapeDtypeStruct((N_TOK, N_EMB), jnp.float32),
            jax.ShapeDtypeStruct((N_TOK, E_DIM), jnp.float32),
            jax.ShapeDtypeStruct((N_TOK, 1), jnp.int32),
            jax.ShapeDtypeStruct((GRID, 1, 1), jnp.float32),
            jax.ShapeDtypeStruct((GRID, 1, N_EMB), jnp.float32),
        ],
        compiler_params=pltpu.CompilerParams(
            dimension_semantics=("parallel",),
        ),
    )(z_r, emb)
    loss, ppl = pl.pallas_call(
        _finish_body,
        out_shape=[
            jax.ShapeDtypeStruct((1, 1), jnp.float32),
            jax.ShapeDtypeStruct((1, 1), jnp.float32),
        ],
    )(psum, pcnt)
    return enc, zq, idx, loss, ppl


def kernel(z, emb):
    z_p = jnp.transpose(z, (0, 2, 3, 1))
    z_flat = z_p.reshape(-1, E_DIM)
    enc, zq_flat, idx, loss, ppl = _vq_call(z_flat, emb)
    z_q = jnp.transpose(zq_flat.reshape(z_p.shape), (0, 2, 3, 1))
    return (loss[0, 0], z_q, ppl[0, 0], enc, idx)


# speedup vs baseline: 1.3455x; 1.0146x over previous
import jax
import jax.numpy as jnp
from jax.experimental import pallas as pl
from jax.experimental.pallas import tpu as pltpu

N_EMB = 1024
E_DIM = 64
COMMIT_COST = 0.25
N_TOK = 32768
ROWS = 4096
GRID = N_TOK // ROWS


def _vq_body(z_ref, emb_ref, enc_ref, zq_ref, idx_ref, psum_ref, pcnt_ref):
    z = z_ref[...]
    emb = emb_ref[...]

    z2 = jnp.sum(z * z, axis=1, keepdims=True)
    e2 = jnp.sum(emb * emb, axis=1, keepdims=True)
    mm2 = jax.lax.dot_general(z, emb + emb, (((1,), (1,)), ((), ())),
                              preferred_element_type=jnp.float32)
    d = (z2 + e2[:, 0][None, :]) - mm2

    dmin = jnp.min(d, axis=1, keepdims=True)
    colsf = jax.lax.broadcasted_iota(jnp.int32, (ROWS, N_EMB), 1).astype(jnp.float32)
    idxf = jnp.min(jnp.where(d == dmin, colsf, float(N_EMB)), axis=1,
                   keepdims=True)

    oh = jnp.where(colsf == idxf, 1.0, 0.0).astype(jnp.float32)
    enc_ref[...] = oh
    zq = jax.lax.dot_general(oh, emb, (((1,), (0,)), ((), ())),
                             preferred_element_type=jnp.float32)
    zq_ref[...] = zq
    idx_ref[...] = idxf.astype(jnp.int32)

    diff = zq - z
    psum_ref[...] = jnp.sum(diff * diff, axis=(0, 1), keepdims=True)[:, :, None]
    ones_row = jnp.full((1, ROWS), 1.0, jnp.float32)
    pcnt_ref[...] = jax.lax.dot_general(ones_row, oh, (((1,), (0,)), ((), ())),
                                        preferred_element_type=jnp.float32)[None]


def _finish_body(psum_ref, pcnt_ref, loss_ref, ppl_ref):
    total = jnp.sum(psum_ref[...], axis=(0, 1, 2))
    mse = total / (N_TOK * E_DIM)
    loss_ref[...] = jnp.full((1, 1), 0.0, jnp.float32) + mse * (1.0 + COMMIT_COST)
    e_mean = jnp.sum(pcnt_ref[...], axis=0) / N_TOK
    ent = -jnp.sum(e_mean * jnp.log(e_mean + 1e-10), axis=(0, 1), keepdims=True)
    ppl_ref[...] = jnp.exp(ent)


def _vq_call(z_r, emb):
    enc, zq, idx, psum, pcnt = pl.pallas_call(
        _vq_body,
        grid=(GRID,),
        in_specs=[
            pl.BlockSpec((ROWS, E_DIM), lambda i: (i, 0)),
            pl.BlockSpec((N_EMB, E_DIM), lambda i: (0, 0)),
        ],
        out_specs=[
            pl.BlockSpec((ROWS, N_EMB), lambda i: (i, 0)),
            pl.BlockSpec((ROWS, E_DIM), lambda i: (i, 0)),
            pl.BlockSpec((ROWS, 1), lambda i: (i, 0)),
            pl.BlockSpec((1, 1, 1), lambda i: (i, 0, 0)),
            pl.BlockSpec((1, 1, N_EMB), lambda i: (i, 0, 0)),
        ],
        out_shape=[
            jax.ShapeDtypeStruct((N_TOK, N_EMB), jnp.float32),
            jax.ShapeDtypeStruct((N_TOK, E_DIM), jnp.float32),
            jax.ShapeDtypeStruct((N_TOK, 1), jnp.int32),
            jax.ShapeDtypeStruct((GRID, 1, 1), jnp.float32),
            jax.ShapeDtypeStruct((GRID, 1, N_EMB), jnp.float32),
        ],
        compiler_params=pltpu.CompilerParams(
            dimension_semantics=("parallel",),
            vmem_limit_bytes=100 * 1024 * 1024,
        ),
    )(z_r, emb)
    loss, ppl = pl.pallas_call(
        _finish_body,
        out_shape=[
            jax.ShapeDtypeStruct((1, 1), jnp.float32),
            jax.ShapeDtypeStruct((1, 1), jnp.float32),
        ],
    )(psum, pcnt)
    return enc, zq, idx, loss, ppl


def kernel(z, emb):
    z_p = jnp.transpose(z, (0, 2, 3, 1))
    z_flat = z_p.reshape(-1, E_DIM)
    enc, zq_flat, idx, loss, ppl = _vq_call(z_flat, emb)
    z_q = jnp.transpose(zq_flat.reshape(z_p.shape), (0, 2, 3, 1))
    return (loss[0, 0], z_q, ppl[0, 0], enc, idx)
